# Initial kernel scaffold; baseline (speedup 1.0000x reference)
import jax
import jax.numpy as jnp
from jax.experimental import pallas as pl


def kernel(x, Wr, Wg, Wu, Wd, Wg_s, Wu_s, Wd_s):
    raise NotImplementedError("write your pallas kernel here")


# speedup vs baseline: 1.0187x; 1.0187x over previous
import functools
import jax
import jax.numpy as jnp
from jax import lax
from jax.experimental import pallas as pl
from jax.experimental.pallas import tpu as pltpu

D_MODEL = 768
D_FF = 1536
N_ROUTED = 7
N_EXP = 8
TOP_K = 2
TILE = 128


def _moe_body(x_ref, wr_ref, wg_ref, wu_ref, wd_ref, out_ref, acc_ref):
    e = pl.program_id(0)
    t = pl.program_id(1)
    x = x_ref[...]

    logits = jax.lax.dot_general(
        x, wr_ref[...], (((1,), (1,)), ((), ())),
        preferred_element_type=jnp.float32)
    col = lax.broadcasted_iota(jnp.int32, logits.shape, 1)
    m1 = jnp.max(logits, axis=1, keepdims=True)
    i1 = jnp.min(jnp.where(logits == m1, col, N_ROUTED), axis=1, keepdims=True)
    l2 = jnp.where(col == i1, -jnp.inf, logits)
    m2 = jnp.max(l2, axis=1, keepdims=True)
    i2 = jnp.min(jnp.where(l2 == m2, col, N_ROUTED), axis=1, keepdims=True)
    w1 = 1.0 / (1.0 + jnp.exp(m2 - m1))
    w2 = 1.0 - w1
    gate = jnp.where(i1 == e, w1, 0.0) + jnp.where(i2 == e, w2, 0.0)
    gate = jnp.where(e == N_ROUTED, 1.0, gate)

    xb = x.astype(jnp.bfloat16)
    g = jax.lax.dot_general(xb, wg_ref[0].astype(jnp.bfloat16),
                            (((1,), (0,)), ((), ())),
                            preferred_element_type=jnp.float32)
    u = jax.lax.dot_general(xb, wu_ref[0].astype(jnp.bfloat16),
                            (((1,), (0,)), ((), ())),
                            preferred_element_type=jnp.float32)
    h = (g / (1.0 + jnp.exp(-g))) * u
    contrib = jax.lax.dot_general(h.astype(jnp.bfloat16),
                                  wd_ref[0].astype(jnp.bfloat16),
                                  (((1,), (0,)), ((), ())),
                                  preferred_element_type=jnp.float32)
    contrib = gate * contrib

    rows = pl.ds(t * TILE, TILE)

    @pl.when(e == 0)
    def _():
        acc_ref[rows, :] = contrib

    @pl.when(e > 0)
    def _():
        acc_ref[rows, :] = acc_ref[rows, :] + contrib

    @pl.when(e == N_EXP - 1)
    def _():
        out_ref[...] = acc_ref[rows, :]


@jax.jit
def kernel(x, Wr, Wg, Wu, Wd, Wg_s, Wu_s, Wd_s):
    orig_shape = x.shape
    flat = x.reshape(-1, D_MODEL)
    n = flat.shape[0]
    wg_all = jnp.concatenate([Wg, Wg_s], axis=0)
    wu_all = jnp.concatenate([Wu, Wu_s], axis=0)
    wd_all = jnp.concatenate([Wd, Wd_s], axis=0)

    grid = (N_EXP, n // TILE)
    out = pl.pallas_call(
        _moe_body,
        grid=grid,
        in_specs=[
            pl.BlockSpec((TILE, D_MODEL), lambda e, t: (t, 0)),
            pl.BlockSpec((N_ROUTED, D_MODEL), lambda e, t: (0, 0)),
            pl.BlockSpec((1, D_MODEL, D_FF), lambda e, t: (e, 0, 0)),
            pl.BlockSpec((1, D_MODEL, D_FF), lambda e, t: (e, 0, 0)),
            pl.BlockSpec((1, D_FF, D_MODEL), lambda e, t: (e, 0, 0)),
        ],
        out_specs=pl.BlockSpec((TILE, D_MODEL), lambda e, t: (t, 0)),
        out_shape=jax.ShapeDtypeStruct((n, D_MODEL), jnp.float32),
        scratch_shapes=[pltpu.VMEM((n, D_MODEL), jnp.float32)],
        compiler_params=pltpu.CompilerParams(
            dimension_semantics=("arbitrary", "arbitrary")),
    )(flat, Wr, wg_all, wu_all, wd_all)
    return out.reshape(orig_shape)


# speedup vs baseline: 1.0776x; 1.0578x over previous
import functools
import jax
import jax.numpy as jnp
from jax import lax
from jax.experimental import pallas as pl
from jax.experimental.pallas import tpu as pltpu

D_MODEL = 768
D_FF = 1536
N_ROUTED = 7
N_EXP = 8
TOP_K = 2
TILE = 128


def _moe_body(x_ref, wr_ref, wg_ref, wu_ref, wd_ref, out_ref, acc_ref):
    e = pl.program_id(0)
    t = pl.program_id(1)
    x = x_ref[...]

    logits = jax.lax.dot_general(
        x, wr_ref[...], (((1,), (1,)), ((), ())),
        preferred_element_type=jnp.float32)
    col = lax.broadcasted_iota(jnp.int32, logits.shape, 1)
    m1 = jnp.max(logits, axis=1, keepdims=True)
    i1 = jnp.min(jnp.where(logits == m1, col, N_ROUTED), axis=1, keepdims=True)
    l2 = jnp.where(col == i1, -jnp.inf, logits)
    m2 = jnp.max(l2, axis=1, keepdims=True)
    i2 = jnp.min(jnp.where(l2 == m2, col, N_ROUTED), axis=1, keepdims=True)
    w1 = 1.0 / (1.0 + jnp.exp(m2 - m1))
    w2 = 1.0 - w1
    gate = jnp.where(i1 == e, w1, 0.0) + jnp.where(i2 == e, w2, 0.0)
    gate = jnp.where(e == N_ROUTED, 1.0, gate)

    xb = x.astype(jnp.bfloat16)
    g = jax.lax.dot_general(xb, wg_ref[0], (((1,), (0,)), ((), ())),
                            preferred_element_type=jnp.float32)
    u = jax.lax.dot_general(xb, wu_ref[0], (((1,), (0,)), ((), ())),
                            preferred_element_type=jnp.float32)
    h = (g / (1.0 + jnp.exp(-g))) * u
    contrib = jax.lax.dot_general(h.astype(jnp.bfloat16), wd_ref[0],
                                  (((1,), (0,)), ((), ())),
                                  preferred_element_type=jnp.float32)
    contrib = gate * contrib

    rows = pl.ds(t * TILE, TILE)

    @pl.when(e == 0)
    def _():
        acc_ref[rows, :] = contrib

    @pl.when(e > 0)
    def _():
        acc_ref[rows, :] = acc_ref[rows, :] + contrib

    @pl.when(e == N_EXP - 1)
    def _():
        out_ref[...] = acc_ref[rows, :]


@jax.jit
def kernel(x, Wr, Wg, Wu, Wd, Wg_s, Wu_s, Wd_s):
    orig_shape = x.shape
    flat = x.reshape(-1, D_MODEL)
    n = flat.shape[0]
    wg_all = jnp.concatenate([Wg, Wg_s], axis=0).astype(jnp.bfloat16)
    wu_all = jnp.concatenate([Wu, Wu_s], axis=0).astype(jnp.bfloat16)
    wd_all = jnp.concatenate([Wd, Wd_s], axis=0).astype(jnp.bfloat16)

    grid = (N_EXP, n // TILE)
    out = pl.pallas_call(
        _moe_body,
        grid=grid,
        in_specs=[
            pl.BlockSpec((TILE, D_MODEL), lambda e, t: (t, 0)),
            pl.BlockSpec((N_ROUTED, D_MODEL), lambda e, t: (0, 0)),
            pl.BlockSpec((1, D_MODEL, D_FF), lambda e, t: (e, 0, 0)),
            pl.BlockSpec((1, D_MODEL, D_FF), lambda e, t: (e, 0, 0)),
            pl.BlockSpec((1, D_FF, D_MODEL), lambda e, t: (e, 0, 0)),
        ],
        out_specs=pl.BlockSpec((TILE, D_MODEL), lambda e, t: (t, 0)),
        out_shape=jax.ShapeDtypeStruct((n, D_MODEL), jnp.float32),
        scratch_shapes=[pltpu.VMEM((n, D_MODEL), jnp.float32)],
        compiler_params=pltpu.CompilerParams(
            dimension_semantics=("arbitrary", "arbitrary")),
    )(flat, Wr, wg_all, wu_all, wd_all)
    return out.reshape(orig_shape)
